# baseline (speedup 1.0000x reference)
import functools

import jax
import jax.numpy as jnp
from jax import lax
from jax.experimental import pallas as pl
from jax.experimental.pallas import tpu as pltpu
from jax.experimental.pallas import tpu_sc as plsc

N = 10000
E = 320000
D_IN = 128
D_HID = 256
D_OUT = 128
EPS = 1e-5

NC = 2
NS = 16
NW = NC * NS
B = 128
NB = 80
CH = 16
NCH = NB // CH
PW = NB * B
EP = NW * PW
N_ACC = 10112
ROWS_PT = N_ACC // NS


def _segment_sum_sc(table, src_p, dst_p, w_p):
    mesh = plsc.VectorSubcoreMesh(core_axis_name="c", subcore_axis_name="s")

    @functools.partial(
        pl.kernel,
        out_type=jax.ShapeDtypeStruct((NC * N_ACC, D_IN), jnp.float32),
        mesh=mesh,
        scratch_types=[
            pltpu.VMEM_SHARED((N_ACC, D_IN), jnp.float32),
            pltpu.VMEM((B, D_IN), jnp.float32),
            pltpu.VMEM((B, D_IN), jnp.float32),
            pltpu.VMEM((2, CH, B), jnp.int32),
            pltpu.VMEM((2, CH, B), jnp.int32),
            pltpu.VMEM((2, CH, B), jnp.float32),
            pltpu.SemaphoreType.DMA,
            pltpu.SemaphoreType.DMA,
            pltpu.SemaphoreType.DMA,
            pltpu.SemaphoreType.DMA,
            pltpu.SemaphoreType.DMA,
        ],
    )
    def seg_kernel(table_h, src_h, dst_h, w_h, out_h, accum, rows0, rows1,
                   sidx, didx, wv, sem_st, sem_g0, sem_g1, sem_s0, sem_s1):
        c = lax.axis_index("c")
        s = lax.axis_index("s")
        wid = s * NC + c

        def start_stage(j, p):
            d0 = pltpu.async_copy(src_h.at[wid, j], sidx.at[p], sem_st)
            d1 = pltpu.async_copy(dst_h.at[wid, j], didx.at[p], sem_st)
            d2 = pltpu.async_copy(w_h.at[wid, j], wv.at[p], sem_st)
            return d0, d1, d2

        def wait_stage(descs):
            for d in descs:
                d.wait()

        st = start_stage(0, 0)

        zero16 = jnp.zeros((16,), jnp.float32)

        def zero_body(b, carry):
            for k in range(D_IN // 16):
                rows0[b, pl.ds(k * 16, 16)] = zero16
            return carry

        lax.fori_loop(0, B, zero_body, 0)

        full, rem = divmod(ROWS_PT, B)
        for j in range(full):
            pltpu.sync_copy(rows0, accum.at[pl.ds(s * ROWS_PT + j * B, B)])
        if rem:
            pltpu.sync_copy(rows0.at[pl.ds(0, rem)],
                            accum.at[pl.ds(s * ROWS_PT + full * B, rem)])
        wait_stage(st)
        plsc.subcore_barrier()

        def scale(rows, p, g):
            def scale_body(g16, carry2):
                wvec = wv[p, g, pl.ds(g16 * 16, 16)]
                for l in range(16):
                    wt = wvec[l]
                    b = g16 * 16 + l
                    for k in range(D_IN // 16):
                        rows[b, pl.ds(k * 16, 16)] = rows[b, pl.ds(k * 16, 16)] * wt
                return carry2

            lax.fori_loop(0, B // 16, scale_body, 0)

        def start_gather(p, g, rows, sem):
            return pltpu.async_copy(table_h.at[sidx.at[p].at[g]], rows, sem)

        def wait_gather(p, g, rows, sem):
            pltpu.make_async_copy(table_h.at[sidx.at[p].at[g]], rows, sem).wait()

        def start_scatter(p, g, rows, sem):
            return pltpu.async_copy(rows, accum.at[didx.at[p].at[g]], sem, add=True)

        def wait_scatter(p, g, rows, sem):
            pltpu.make_async_copy(rows, accum.at[didx.at[p].at[g]], sem).wait()

        start_gather(0, 0, rows0, sem_g0)
        start_gather(0, 1, rows1, sem_g1)
        npairs = CH // 2

        for j in range(NCH):
            p = j % 2
            q = 1 - p
            if j + 1 < NCH:
                st = start_stage(j + 1, q)

            def batch_body(k, carry, p=p, q=q, last_chunk=(j + 1 == NCH)):
                a = 2 * k
                b = a + 1
                wait_gather(p, a, rows0, sem_g0)
                scale(rows0, p, a)
                start_scatter(p, a, rows0, sem_s0)
                wait_gather(p, b, rows1, sem_g1)
                scale(rows1, p, b)
                start_scatter(p, b, rows1, sem_s1)
                wait_scatter(p, a, rows0, sem_s0)

                @pl.when(k < npairs - 1)
                def _():
                    start_gather(p, a + 2, rows0, sem_g0)

                if not last_chunk:
                    @pl.when(k == npairs - 1)
                    def _():
                        wait_stage(st)
                        start_gather(q, 0, rows0, sem_g0)

                wait_scatter(p, b, rows1, sem_s1)

                @pl.when(k < npairs - 1)
                def _():
                    start_gather(p, b + 2, rows1, sem_g1)

                if not last_chunk:
                    @pl.when(k == npairs - 1)
                    def _():
                        start_gather(q, 1, rows1, sem_g1)

                return carry

            lax.fori_loop(0, npairs, batch_body, 0)

        plsc.subcore_barrier()

        pltpu.sync_copy(accum.at[pl.ds(s * ROWS_PT, ROWS_PT)],
                        out_h.at[pl.ds(c * N_ACC + s * ROWS_PT, ROWS_PT)])

    return seg_kernel(table, src_p, dst_p, w_p)


def _dense_stage(partials, x, W1_rel, b1, W1_root, gamma1, beta1, W2_rel, b2, W2_root):
    def body(p_ref, x_ref, w1r_ref, b1_ref, w1o_ref, g1_ref, be1_ref,
             w2r_ref, b2_ref, w2o_ref, hp_ref, root2_ref):
        agg = p_ref[0] + p_ref[1]
        h = lax.dot_general(agg, w1r_ref[...], (((1,), (1,)), ((), ())),
                            preferred_element_type=jnp.float32)
        h = h + lax.dot_general(x_ref[...], w1o_ref[...], (((1,), (1,)), ((), ())),
                                preferred_element_type=jnp.float32)
        h = h + b1_ref[...]
        mean = jnp.mean(h, axis=0, keepdims=True)
        var = jnp.mean((h - mean) ** 2, axis=0, keepdims=True)
        hn = (h - mean) * lax.rsqrt(var + EPS) * g1_ref[...] + be1_ref[...]
        hn = jnp.maximum(hn, 0.0)
        hp_ref[...] = lax.dot_general(hn, w2r_ref[...], (((1,), (1,)), ((), ())),
                                      preferred_element_type=jnp.float32)
        root2_ref[...] = lax.dot_general(hn, w2o_ref[...], (((1,), (1,)), ((), ())),
                                         preferred_element_type=jnp.float32) + b2_ref[...]

    return pl.pallas_call(
        body,
        out_shape=[
            jax.ShapeDtypeStruct((N, D_OUT), jnp.float32),
            jax.ShapeDtypeStruct((N, D_OUT), jnp.float32),
        ],
    )(partials, x, W1_rel, b1.reshape(1, D_HID), W1_root,
      gamma1.reshape(1, D_HID), beta1.reshape(1, D_HID), W2_rel,
      b2.reshape(1, D_OUT), W2_root)


def _final_add(partials, root2):
    def body(p_ref, r_ref, o_ref):
        o_ref[...] = p_ref[0] + p_ref[1] + r_ref[...]

    return pl.pallas_call(
        body,
        out_shape=jax.ShapeDtypeStruct((N, D_OUT), jnp.float32),
    )(partials, root2)


def kernel(x1, edge_index, edge_weight, W1_rel, b1_rel, W1_root, gamma1, beta1,
           W2_rel, b2_rel, W2_root):
    src = edge_index[0]
    dst = edge_index[1]
    pad = EP - E
    src_p = jnp.concatenate([src, jnp.zeros((pad,), jnp.int32)]).reshape(NW, NCH, CH, B)
    dst_p = jnp.concatenate([dst, jnp.zeros((pad,), jnp.int32)]).reshape(NW, NCH, CH, B)
    w_p = jnp.concatenate([edge_weight, jnp.zeros((pad,), jnp.float32)]).reshape(NW, NCH, CH, B)

    p1 = _segment_sum_sc(x1, src_p, dst_p, w_p).reshape(NC, N_ACC, D_IN)[:, :N]
    hp, root2 = _dense_stage(p1, x1, W1_rel, b1_rel, W1_root, gamma1, beta1,
                             W2_rel, b2_rel, W2_root)
    p2 = _segment_sum_sc(hp, src_p, dst_p, w_p).reshape(NC, N_ACC, D_IN)[:, :N]
    return _final_add(p2, root2)
